# Initial kernel scaffold; baseline (speedup 1.0000x reference)
import jax
import jax.numpy as jnp
from jax.experimental import pallas as pl


def kernel(f, edge_src, edge_x, Wq, bq, Wk, bk, Wv, bv, Wo, bo, W_mp0, b_mp0, W_last, b_last):
    raise NotImplementedError("write your pallas kernel here")


# speedup vs baseline: 4.3653x; 4.3653x over previous
import functools

import jax
import jax.numpy as jnp
from jax import lax
from jax.experimental import pallas as pl
from jax.experimental.pallas import tpu as pltpu
from jax.experimental.pallas import tpu_sc as plsc

N = 10000
DEG = 32
E = N * DEG
HID = 128
HEADS = 4
DK = HID // HEADS

B_A = 200
B_B = 200

NW = 32
PER_W = E // NW
CH = 80
NCH = PER_W // CH


def _attn_body(f_ref, ex_ref, wq_ref, bq_ref, wk_ref, bk_ref, wv_ref, bv_ref,
               wo_ref, bo_ref, sel_ref, selt_ref, fh_ref):
    b = f_ref.shape[0]
    fb = f_ref[...]
    ex = ex_ref[...]
    q = jnp.dot(fb, wq_ref[...], preferred_element_type=jnp.float32) + bq_ref[...]
    k = jnp.dot(ex, wk_ref[...], preferred_element_type=jnp.float32) + bk_ref[...]
    v = jnp.dot(ex, wv_ref[...], preferred_element_type=jnp.float32) + bv_ref[...]
    k3 = k.reshape(b, DEG, HID)
    qk = (k3 * q[:, None, :]).reshape(b * DEG, HID)
    s = jnp.dot(qk, sel_ref[...], preferred_element_type=jnp.float32) * (DK ** -0.5)
    s3 = s.reshape(b, DEG, HEADS)
    s3 = s3 - jnp.max(s3, axis=1, keepdims=True)
    e3 = jnp.exp(s3)
    p3 = e3 / jnp.sum(e3, axis=1, keepdims=True)
    pf = jnp.dot(p3.reshape(b * DEG, HEADS), selt_ref[...],
                 preferred_element_type=jnp.float32)
    x = jnp.sum((pf * v).reshape(b, DEG, HID), axis=1)
    attn = jnp.dot(x, wo_ref[...], preferred_element_type=jnp.float32) + bo_ref[...]
    fh_ref[...] = attn + fb


def _pairswap(x):
    up = jnp.concatenate([x[1:], x[:1]], axis=0)
    dn = jnp.concatenate([x[-1:], x[:-1]], axis=0)
    par = lax.broadcasted_iota(jnp.int32, x.shape, 0) % 2
    return jnp.where(par == 0, up, dn)


def _edge_body(ex_ref, g_ref, fh_ref, f_ref, wmp_ref, bmp_ref,
               w1_ref, w2_ref, w3_ref, bl_ref, out_ref):
    b = fh_ref.shape[0]
    ex = ex_ref[...]
    g = g_ref[...]
    rev = _pairswap(ex)
    t = jnp.dot(g - rev, wmp_ref[...], preferred_element_type=jnp.float32) + bmp_ref[...]
    h = jnp.maximum(ex + t, 0.0)
    ms = jnp.sum(h.reshape(b, DEG, HID), axis=1)
    out = (jnp.dot(ms, w1_ref[...], preferred_element_type=jnp.float32)
           + jnp.dot(fh_ref[...], w2_ref[...], preferred_element_type=jnp.float32)
           + jnp.dot(f_ref[...], w3_ref[...], preferred_element_type=jnp.float32)
           + bl_ref[...])
    out_ref[...] = out


def _full(shape):
    return pl.BlockSpec(shape, lambda i: (0, 0))


@functools.lru_cache(maxsize=1)
def _sc_gather_fn():
    mesh = plsc.VectorSubcoreMesh(core_axis_name="c", subcore_axis_name="s")

    @functools.partial(
        pl.kernel,
        mesh=mesh,
        out_type=jax.ShapeDtypeStruct((E, HID), jnp.float32),
        scratch_types=[
            pltpu.VMEM((NCH, CH), jnp.int32),
            pltpu.VMEM((CH, HID), jnp.float32),
            pltpu.SemaphoreType.DMA,
        ],
    )
    def _sc_gather(table_hbm, idx_hbm, out_hbm, idx_v, buf, sem):
        w = lax.axis_index("s") * 2 + lax.axis_index("c")
        pltpu.sync_copy(idx_hbm.at[w], idx_v)
        base = w * PER_W

        def body(j, carry):
            pltpu.async_copy(table_hbm.at[idx_v.at[j]], buf, sem).wait()
            pltpu.sync_copy(buf, out_hbm.at[pl.ds(base + j * CH, CH)])
            return carry

        lax.fori_loop(0, NCH, body, 0)

    return _sc_gather


def kernel(f, edge_src, edge_x, Wq, bq, Wk, bk, Wv, bv, Wo, bo,
           W_mp0, b_mp0, W_last, b_last):
    wqT, wkT, wvT, woT, wmpT = Wq.T, Wk.T, Wv.T, Wo.T, W_mp0.T
    wlT = W_last.T
    w1, w2, w3 = wlT[:HID], wlT[HID:2 * HID], wlT[2 * HID:]
    sel = (jnp.arange(HID)[:, None] // DK
           == jnp.arange(HEADS)[None, :]).astype(jnp.float32)
    selt = sel.T
    bq2, bk2, bv2, bo2 = bq[None], bk[None], bv[None], bo[None]
    bmp2, bl2 = b_mp0[None], b_last[None]

    fh = pl.pallas_call(
        _attn_body,
        grid=(N // B_A,),
        in_specs=[
            pl.BlockSpec((B_A, HID), lambda i: (i, 0)),
            pl.BlockSpec((B_A * DEG, HID), lambda i: (i, 0)),
            _full((HID, HID)), _full((1, HID)),
            _full((HID, HID)), _full((1, HID)),
            _full((HID, HID)), _full((1, HID)),
            _full((HID, HID)), _full((1, HID)),
            _full((HID, HEADS)), _full((HEADS, HID)),
        ],
        out_specs=pl.BlockSpec((B_A, HID), lambda i: (i, 0)),
        out_shape=jax.ShapeDtypeStruct((N, HID), jnp.float32),
    )(f, edge_x, wqT, bq2, wkT, bk2, wvT, bv2, woT, bo2, sel, selt)

    idx3 = edge_src.reshape(NW, NCH, CH)
    g = _sc_gather_fn()(fh, idx3)

    out = pl.pallas_call(
        _edge_body,
        grid=(N // B_B,),
        in_specs=[
            pl.BlockSpec((B_B * DEG, HID), lambda i: (i, 0)),
            pl.BlockSpec((B_B * DEG, HID), lambda i: (i, 0)),
            pl.BlockSpec((B_B, HID), lambda i: (i, 0)),
            pl.BlockSpec((B_B, HID), lambda i: (i, 0)),
            _full((HID, HID)), _full((1, HID)),
            _full((HID, HID)), _full((HID, HID)), _full((HID, HID)),
            _full((1, HID)),
        ],
        out_specs=pl.BlockSpec((B_B, HID), lambda i: (i, 0)),
        out_shape=jax.ShapeDtypeStruct((N, HID), jnp.float32),
    )(edge_x, g, fh, f, wmpT, bmp2, w1, w2, w3, bl2)
    return out
